# Initial kernel scaffold; baseline (speedup 1.0000x reference)
import jax
import jax.numpy as jnp
from jax.experimental import pallas as pl


def kernel(p, x, o, W1, bn1_g, bn1_b, Wq, bq, Wk, bk, Wv, bv, Wp1, bp1, bnp_g, bnp_b, Wp2, bp2, bnw1_g, bnw1_b, Ww1, bw1, bnw2_g, bnw2_b, Ww2, bw2, bn2_g, bn2_b, W3, bn3_g, bn3_b):
    raise NotImplementedError("write your pallas kernel here")


# speedup vs baseline: 2.0723x; 2.0723x over previous
import functools

import jax
import jax.numpy as jnp
from jax import lax
from jax.experimental import pallas as pl
from jax.experimental.pallas import tpu as pltpu
from jax.experimental.pallas import tpu_sc as plsc

N = 10000
C = 256
K = 8
S = 8
CS = 32
EPS = 1e-5

RB = 200
NRB = N // RB
CW = 1000
NCT = N // CW
GB = RB * K
B_PAD = 81920
CHUNK = 128

_F32 = jnp.float32
_BIG = 2**30


def _affine(s, q, count, g, b):
    m = s / count
    v = q / count - m * m
    a = g * lax.rsqrt(v + EPS)
    return a, b - m * a


def _k1_body(x_ref, w_ref, y_ref, s_ref, q_ref):
    i = pl.program_id(0)
    y = jnp.dot(x_ref[...], w_ref[...], preferred_element_type=_F32)
    y_ref[...] = y

    @pl.when(i == 0)
    def _():
        s_ref[...] = jnp.zeros_like(s_ref)
        q_ref[...] = jnp.zeros_like(q_ref)

    s_ref[...] += jnp.sum(y, axis=0, keepdims=True)
    q_ref[...] += jnp.sum(y * y, axis=0, keepdims=True)


def _k1(x, W1):
    return pl.pallas_call(
        _k1_body,
        grid=(NRB,),
        in_specs=[
            pl.BlockSpec((RB, C), lambda i: (i, 0)),
            pl.BlockSpec((C, C), lambda i: (0, 0)),
        ],
        out_specs=[
            pl.BlockSpec((RB, C), lambda i: (i, 0)),
            pl.BlockSpec((1, C), lambda i: (0, 0)),
            pl.BlockSpec((1, C), lambda i: (0, 0)),
        ],
        out_shape=[
            jax.ShapeDtypeStruct((N, C), _F32),
            jax.ShapeDtypeStruct((1, C), _F32),
            jax.ShapeDtypeStruct((1, C), _F32),
        ],
    )(x, W1)


def _k2_body(y_ref, a_ref, c_ref, wq_ref, bq_ref, wk_ref, bk_ref,
             wv_ref, bv_ref, xq_ref, xk_ref, xv_ref):
    x1 = jnp.maximum(y_ref[...] * a_ref[...] + c_ref[...], 0.0)
    xq_ref[...] = jnp.dot(x1, wq_ref[...], preferred_element_type=_F32) + bq_ref[...]
    xk_ref[...] = jnp.dot(x1, wk_ref[...], preferred_element_type=_F32) + bk_ref[...]
    xv_ref[...] = jnp.dot(x1, wv_ref[...], preferred_element_type=_F32) + bv_ref[...]


def _k2(y1, a1, c1, Wq, bq, Wk, bk, Wv, bv):
    row = pl.BlockSpec((RB, C), lambda i: (i, 0))
    cst = pl.BlockSpec((1, C), lambda i: (0, 0))
    mat = pl.BlockSpec((C, C), lambda i: (0, 0))
    return pl.pallas_call(
        _k2_body,
        grid=(NRB,),
        in_specs=[row, cst, cst, mat, cst, mat, cst, mat, cst],
        out_specs=[row, row, row],
        out_shape=[jax.ShapeDtypeStruct((N, C), _F32)] * 3,
    )(y1, a1, c1, Wq, bq.reshape(1, C), Wk, bk.reshape(1, C), Wv, bv.reshape(1, C))


def _k3_body(pr_ref, pc_ref, idx_ref, bd_ref):
    j = pl.program_id(1)
    prow = pr_ref[...]
    pcol = pc_ref[...]
    dot = lax.dot_general(prow.astype(jnp.bfloat16), pcol.astype(jnp.bfloat16),
                          (((1,), (1,)), ((), ())),
                          preferred_element_type=_F32)
    pn_r = jnp.sum(prow * prow, axis=1, keepdims=True)
    pn_c = lax.dot_general(jnp.ones((1, 16), _F32), pcol * pcol,
                           (((1,), (1,)), ((), ())), preferred_element_type=_F32,
                           precision=lax.Precision.HIGHEST)
    d2 = pn_r + pn_c - 2.0 * dot

    @pl.when(j == 0)
    def _():
        bd_ref[...] = jnp.full((RB, K), jnp.inf, _F32)
        idx_ref[...] = jnp.zeros((RB, K), jnp.int32)

    lane = lax.broadcasted_iota(jnp.int32, (RB, CW), 1)
    d = d2
    cds, cis = [], []
    for _ in range(K):
        m = jnp.min(d, axis=1, keepdims=True)
        pos = jnp.min(jnp.where(d == m, lane, _BIG), axis=1, keepdims=True)
        cds.append(m)
        cis.append(pos + j * CW)
        d = jnp.where(lane == pos, jnp.inf, d)
    comb_d = jnp.concatenate(cds + [bd_ref[...]], axis=1)
    comb_i = jnp.concatenate(cis + [idx_ref[...]], axis=1)
    lane16 = lax.broadcasted_iota(jnp.int32, (RB, 2 * K), 1)
    nds, nis = [], []
    for _ in range(K):
        m = jnp.min(comb_d, axis=1, keepdims=True)
        pos = jnp.min(jnp.where(comb_d == m, lane16, _BIG), axis=1, keepdims=True)
        vi = jnp.sum(jnp.where(lane16 == pos, comb_i, 0), axis=1, keepdims=True)
        nds.append(m)
        nis.append(vi)
        comb_d = jnp.where(lane16 == pos, jnp.inf, comb_d)
    bd_ref[...] = jnp.concatenate(nds, axis=1)
    idx_ref[...] = jnp.concatenate(nis, axis=1)


def _k3(p_pad):
    return pl.pallas_call(
        _k3_body,
        grid=(NRB, NCT),
        in_specs=[
            pl.BlockSpec((RB, 16), lambda i, j: (i, 0)),
            pl.BlockSpec((CW, 16), lambda i, j: (j, 0)),
        ],
        out_specs=pl.BlockSpec((RB, K), lambda i, j: (i, 0)),
        out_shape=jax.ShapeDtypeStruct((N, K), jnp.int32),
        scratch_shapes=[pltpu.VMEM((RB, K), _F32)],
    )(p_pad, p_pad)


def _gather_rows(table, idx_flat, D):
    info = plsc.get_sparse_core_info()
    nw = info.num_cores * info.num_subcores
    b_per_w = B_PAD // nw
    nch = b_per_w // CHUNK

    mesh = plsc.VectorSubcoreMesh(core_axis_name="c", subcore_axis_name="s")

    @functools.partial(
        pl.kernel, mesh=mesh,
        out_type=jax.ShapeDtypeStruct((B_PAD, D), _F32),
        scratch_types=[
            pltpu.VMEM((nch, CHUNK), jnp.int32),
            pltpu.VMEM((2, CHUNK, D), _F32),
            pltpu.SemaphoreType.DMA,
            pltpu.SemaphoreType.DMA,
        ],
    )
    def gk(table_hbm, idx_hbm, out_hbm, idx_v, bufs, sem0, sem1):
        wid = lax.axis_index("s") * info.num_cores + lax.axis_index("c")
        base = wid * b_per_w
        for c in range(nch):
            pltpu.sync_copy(idx_hbm.at[pl.ds(base + c * CHUNK, CHUNK)], idx_v.at[c])
        sems = [sem0, sem1]
        prev = None
        for c in range(nch):
            cur = pltpu.async_copy(table_hbm.at[idx_v.at[c]], bufs.at[c % 2],
                                   sems[c % 2])
            if prev is not None:
                pcp, pc = prev
                pcp.wait()
                pltpu.sync_copy(bufs.at[pc % 2],
                                out_hbm.at[pl.ds(base + pc * CHUNK, CHUNK)])
            prev = (cur, c)
        pcp, pc = prev
        pcp.wait()
        pltpu.sync_copy(bufs.at[pc % 2], out_hbm.at[pl.ds(base + pc * CHUNK, CHUNK)])

    return gk(table, idx_flat)


def _pr_block(pg, pblk, w1p, b1p, ap, cp, w2p, b2p):
    d = (pg[:, :16].reshape(RB, K, 16) - pblk[:, None, :]).reshape(GB, 16)
    h = jnp.dot(d, w1p, preferred_element_type=_F32) + b1p
    h = jnp.maximum(h * ap + cp, 0.0)
    return jnp.dot(h, w2p, preferred_element_type=_F32) + b2p


def _k45_body(pg_ref, p_ref, w1p_ref, b1p_ref, s_ref, q_ref):
    i = pl.program_id(0)
    d = (pg_ref[...][:, :16].reshape(RB, K, 16)
         - p_ref[...][:, None, :]).reshape(GB, 16)
    pr1 = jnp.dot(d, w1p_ref[...], preferred_element_type=_F32) + b1p_ref[...]

    @pl.when(i == 0)
    def _():
        s_ref[...] = jnp.zeros_like(s_ref)
        q_ref[...] = jnp.zeros_like(q_ref)

    s_ref[...] += jnp.sum(pr1, axis=0, keepdims=True)
    q_ref[...] += jnp.sum(pr1 * pr1, axis=0, keepdims=True)


def _k45(pg, p_pad, w1p, b1p):
    return pl.pallas_call(
        _k45_body,
        grid=(NRB,),
        in_specs=[
            pl.BlockSpec((GB, 128), lambda i: (i, 0)),
            pl.BlockSpec((RB, 16), lambda i: (i, 0)),
            pl.BlockSpec((16, 16), lambda i: (0, 0)),
            pl.BlockSpec((1, 16), lambda i: (0, 0)),
        ],
        out_specs=[pl.BlockSpec((1, 16), lambda i: (0, 0))] * 2,
        out_shape=[jax.ShapeDtypeStruct((1, 16), _F32)] * 2,
    )(pg, p_pad, w1p, b1p)


def _k5_body(xkg_ref, pg_ref, p_ref, xq_ref, w1p_ref, b1p_ref, ap_ref, cp_ref,
             w2p_ref, b2p_ref, w0_ref, s_ref, q_ref):
    i = pl.program_id(0)
    pr = _pr_block(pg_ref[...], p_ref[...], w1p_ref[...], b1p_ref[...],
                   ap_ref[...], cp_ref[...], w2p_ref[...], b2p_ref[...])
    w0 = (xkg_ref[...].reshape(RB, K, C) - xq_ref[...][:, None, :]
          + pr.reshape(RB, K, C)).reshape(GB, C)
    w0_ref[...] = w0

    @pl.when(i == 0)
    def _():
        s_ref[...] = jnp.zeros_like(s_ref)
        q_ref[...] = jnp.zeros_like(q_ref)

    s_ref[...] += jnp.sum(w0, axis=0, keepdims=True)
    q_ref[...] += jnp.sum(w0 * w0, axis=0, keepdims=True)


def _k5(xkg, pg, p_pad, xq, w1p, b1p, ap, cp, w2p, b2p):
    return pl.pallas_call(
        _k5_body,
        grid=(NRB,),
        in_specs=[
            pl.BlockSpec((GB, C), lambda i: (i, 0)),
            pl.BlockSpec((GB, 128), lambda i: (i, 0)),
            pl.BlockSpec((RB, 16), lambda i: (i, 0)),
            pl.BlockSpec((RB, C), lambda i: (i, 0)),
            pl.BlockSpec((16, 16), lambda i: (0, 0)),
            pl.BlockSpec((1, 16), lambda i: (0, 0)),
            pl.BlockSpec((1, 16), lambda i: (0, 0)),
            pl.BlockSpec((1, 16), lambda i: (0, 0)),
            pl.BlockSpec((16, C), lambda i: (0, 0)),
            pl.BlockSpec((1, C), lambda i: (0, 0)),
        ],
        out_specs=[
            pl.BlockSpec((GB, C), lambda i: (i, 0)),
            pl.BlockSpec((1, C), lambda i: (0, 0)),
            pl.BlockSpec((1, C), lambda i: (0, 0)),
        ],
        out_shape=[
            jax.ShapeDtypeStruct((B_PAD, C), _F32),
            jax.ShapeDtypeStruct((1, C), _F32),
            jax.ShapeDtypeStruct((1, C), _F32),
        ],
    )(xkg, pg, p_pad, xq, w1p, b1p, ap, cp, w2p, b2p)


def _k6_body(w0_ref, a_ref, c_ref, w_ref, b_ref, w1_ref, s_ref, q_ref):
    i = pl.program_id(0)
    h = jnp.maximum(w0_ref[...] * a_ref[...] + c_ref[...], 0.0)
    w1 = jnp.dot(h, w_ref[...], preferred_element_type=_F32) + b_ref[...]
    w1_ref[...] = w1

    @pl.when(i == 0)
    def _():
        s_ref[...] = jnp.zeros_like(s_ref)
        q_ref[...] = jnp.zeros_like(q_ref)

    s_ref[...] += jnp.sum(w1, axis=0, keepdims=True)
    q_ref[...] += jnp.sum(w1 * w1, axis=0, keepdims=True)


def _k6(w0, aw, cw, Ww1, bw1):
    return pl.pallas_call(
        _k6_body,
        grid=(NRB,),
        in_specs=[
            pl.BlockSpec((GB, C), lambda i: (i, 0)),
            pl.BlockSpec((1, C), lambda i: (0, 0)),
            pl.BlockSpec((1, C), lambda i: (0, 0)),
            pl.BlockSpec((C, CS), lambda i: (0, 0)),
            pl.BlockSpec((1, CS), lambda i: (0, 0)),
        ],
        out_specs=[
            pl.BlockSpec((GB, CS), lambda i: (i, 0)),
            pl.BlockSpec((1, CS), lambda i: (0, 0)),
            pl.BlockSpec((1, CS), lambda i: (0, 0)),
        ],
        out_shape=[
            jax.ShapeDtypeStruct((N * K, CS), _F32),
            jax.ShapeDtypeStruct((1, CS), _F32),
            jax.ShapeDtypeStruct((1, CS), _F32),
        ],
    )(w0, aw, cw, Ww1, bw1.reshape(1, CS))


def _k7_body(w1_ref, xvg_ref, pg_ref, p_ref, a2_ref, c2_ref, ww2_ref, bw2_ref,
             w1p_ref, b1p_ref, ap_ref, cp_ref, w2p_ref, b2p_ref,
             agg_ref, s_ref, q_ref):
    i = pl.program_id(0)
    h = jnp.maximum(w1_ref[...] * a2_ref[...] + c2_ref[...], 0.0)
    w2 = jnp.dot(h, ww2_ref[...], preferred_element_type=_F32) + bw2_ref[...]
    w3 = w2.reshape(RB, K, CS)
    m = jnp.max(w3, axis=1, keepdims=True)
    e = jnp.exp(w3 - m)
    sm = e / jnp.sum(e, axis=1, keepdims=True)
    pr = _pr_block(pg_ref[...], p_ref[...], w1p_ref[...], b1p_ref[...],
                   ap_ref[...], cp_ref[...], w2p_ref[...], b2p_ref[...])
    v0 = xvg_ref[...].reshape(RB, K, C) + pr.reshape(RB, K, C)
    wrep = jnp.concatenate([sm] * S, axis=2)
    agg = jnp.sum(v0 * wrep, axis=1)
    agg_ref[...] = agg

    @pl.when(i == 0)
    def _():
        s_ref[...] = jnp.zeros_like(s_ref)
        q_ref[...] = jnp.zeros_like(q_ref)

    s_ref[...] += jnp.sum(agg, axis=0, keepdims=True)
    q_ref[...] += jnp.sum(agg * agg, axis=0, keepdims=True)


def _k7(w1, xvg, pg, p_pad, a2, c2, Ww2, bw2, w1p, b1p, ap, cp, w2p, b2p):
    return pl.pallas_call(
        _k7_body,
        grid=(NRB,),
        in_specs=[
            pl.BlockSpec((GB, CS), lambda i: (i, 0)),
            pl.BlockSpec((GB, C), lambda i: (i, 0)),
            pl.BlockSpec((GB, 128), lambda i: (i, 0)),
            pl.BlockSpec((RB, 16), lambda i: (i, 0)),
            pl.BlockSpec((1, CS), lambda i: (0, 0)),
            pl.BlockSpec((1, CS), lambda i: (0, 0)),
            pl.BlockSpec((CS, CS), lambda i: (0, 0)),
            pl.BlockSpec((1, CS), lambda i: (0, 0)),
            pl.BlockSpec((16, 16), lambda i: (0, 0)),
            pl.BlockSpec((1, 16), lambda i: (0, 0)),
            pl.BlockSpec((1, 16), lambda i: (0, 0)),
            pl.BlockSpec((1, 16), lambda i: (0, 0)),
            pl.BlockSpec((16, C), lambda i: (0, 0)),
            pl.BlockSpec((1, C), lambda i: (0, 0)),
        ],
        out_specs=[
            pl.BlockSpec((RB, C), lambda i: (i, 0)),
            pl.BlockSpec((1, C), lambda i: (0, 0)),
            pl.BlockSpec((1, C), lambda i: (0, 0)),
        ],
        out_shape=[
            jax.ShapeDtypeStruct((N, C), _F32),
            jax.ShapeDtypeStruct((1, C), _F32),
            jax.ShapeDtypeStruct((1, C), _F32),
        ],
    )(w1, xvg, pg, p_pad, a2, c2, Ww2, bw2.reshape(1, CS),
      w1p, b1p, ap, cp, w2p, b2p)


def _k8_body(agg_ref, a_ref, c_ref, w_ref, y_ref, s_ref, q_ref):
    i = pl.program_id(0)
    x2 = jnp.maximum(agg_ref[...] * a_ref[...] + c_ref[...], 0.0)
    y3 = jnp.dot(x2, w_ref[...], preferred_element_type=_F32)
    y_ref[...] = y3

    @pl.when(i == 0)
    def _():
        s_ref[...] = jnp.zeros_like(s_ref)
        q_ref[...] = jnp.zeros_like(q_ref)

    s_ref[...] += jnp.sum(y3, axis=0, keepdims=True)
    q_ref[...] += jnp.sum(y3 * y3, axis=0, keepdims=True)


def _k8(agg, a2, c2, W3):
    row = pl.BlockSpec((RB, C), lambda i: (i, 0))
    cst = pl.BlockSpec((1, C), lambda i: (0, 0))
    return pl.pallas_call(
        _k8_body,
        grid=(NRB,),
        in_specs=[row, cst, cst, pl.BlockSpec((C, C), lambda i: (0, 0))],
        out_specs=[row, cst, cst],
        out_shape=[
            jax.ShapeDtypeStruct((N, C), _F32),
            jax.ShapeDtypeStruct((1, C), _F32),
            jax.ShapeDtypeStruct((1, C), _F32),
        ],
    )(agg, a2, c2, W3)


def _k9_body(y_ref, x_ref, a_ref, c_ref, o_ref):
    o_ref[...] = jnp.maximum(y_ref[...] * a_ref[...] + c_ref[...] + x_ref[...], 0.0)


def _k9(y3, x, a3, c3):
    row = pl.BlockSpec((RB, C), lambda i: (i, 0))
    cst = pl.BlockSpec((1, C), lambda i: (0, 0))
    return pl.pallas_call(
        _k9_body,
        grid=(NRB,),
        in_specs=[row, row, cst, cst],
        out_specs=row,
        out_shape=jax.ShapeDtypeStruct((N, C), _F32),
    )(y3, x, a3, c3)


def kernel(p, x, o, W1, bn1_g, bn1_b, Wq, bq, Wk, bk, Wv, bv, Wp1, bp1,
           bnp_g, bnp_b, Wp2, bp2, bnw1_g, bnw1_b, Ww1, bw1, bnw2_g, bnw2_b,
           Ww2, bw2, bn2_g, bn2_b, W3, bn3_g, bn3_b):
    p_pad = jnp.pad(p, ((0, 0), (0, 13)))

    y1, s1, q1 = _k1(x, W1)
    a1, c1 = _affine(s1, q1, N, bn1_g, bn1_b)
    xq, xk, xv = _k2(y1, a1, c1, Wq, bq, Wk, bk, Wv, bv)

    idx = _k3(p_pad)
    idx_flat = jnp.pad(idx.reshape(-1), (0, B_PAD - N * K))

    xkg = _gather_rows(xk, idx_flat, C)
    xvg = _gather_rows(xv, idx_flat, C)
    pg = _gather_rows(jnp.pad(p, ((0, 0), (0, 125))), idx_flat, 128)

    w1p = jnp.zeros((16, 16), _F32).at[:3, :3].set(Wp1)
    b1p = jnp.zeros((1, 16), _F32).at[0, :3].set(bp1)
    gp = jnp.ones((16,), _F32).at[:3].set(bnp_g)
    bp = jnp.zeros((16,), _F32).at[:3].set(bnp_b)
    w2p = jnp.zeros((16, C), _F32).at[:3, :].set(Wp2)
    b2p = bp2.reshape(1, C)

    sp, qp = _k45(pg, p_pad, w1p, b1p)
    ap, cp = _affine(sp, qp, N * K, gp.reshape(1, 16), bp.reshape(1, 16))

    w0, sw0, qw0 = _k5(xkg, pg, p_pad, xq, w1p, b1p, ap, cp, w2p, b2p)
    aw0, cw0 = _affine(sw0, qw0, N * K, bnw1_g.reshape(1, C), bnw1_b.reshape(1, C))

    w1a, sw1, qw1 = _k6(w0, aw0, cw0, Ww1, bw1)
    aw1, cw1 = _affine(sw1, qw1, N * K, bnw2_g.reshape(1, CS), bnw2_b.reshape(1, CS))

    agg, sa, qa = _k7(w1a, xvg, pg, p_pad, aw1, cw1, Ww2, bw2,
                      w1p, b1p, ap, cp, w2p, b2p)
    a2, c2 = _affine(sa, qa, N, bn2_g.reshape(1, C), bn2_b.reshape(1, C))

    y3, s3, q3 = _k8(agg, a2, c2, W3)
    a3, c3 = _affine(s3, q3, N, bn3_g.reshape(1, C), bn3_b.reshape(1, C))
    return _k9(y3, x, a3, c3)


# speedup vs baseline: 2.3525x; 1.1352x over previous
import functools

import jax
import jax.numpy as jnp
from jax import lax
from jax.experimental import pallas as pl
from jax.experimental.pallas import tpu as pltpu
from jax.experimental.pallas import tpu_sc as plsc

N = 10000
C = 256
K = 8
S = 8
CS = 32
EPS = 1e-5

RB = 200
NRB = N // RB
CW = 1000
NCT = N // CW
GB = RB * K
B_PAD = 81920
CHUNK = 128

_F32 = jnp.float32
_BIG = 2**30


def _affine(s, q, count, g, b):
    m = s / count
    v = q / count - m * m
    a = g * lax.rsqrt(v + EPS)
    return a, b - m * a


def _k1_body(x_ref, w_ref, y_ref, s_ref, q_ref):
    i = pl.program_id(0)
    y = jnp.dot(x_ref[...], w_ref[...], preferred_element_type=_F32)
    y_ref[...] = y

    @pl.when(i == 0)
    def _():
        s_ref[...] = jnp.zeros_like(s_ref)
        q_ref[...] = jnp.zeros_like(q_ref)

    s_ref[...] += jnp.sum(y, axis=0, keepdims=True)
    q_ref[...] += jnp.sum(y * y, axis=0, keepdims=True)


def _k1(x, W1):
    return pl.pallas_call(
        _k1_body,
        grid=(NRB,),
        in_specs=[
            pl.BlockSpec((RB, C), lambda i: (i, 0)),
            pl.BlockSpec((C, C), lambda i: (0, 0)),
        ],
        out_specs=[
            pl.BlockSpec((RB, C), lambda i: (i, 0)),
            pl.BlockSpec((1, C), lambda i: (0, 0)),
            pl.BlockSpec((1, C), lambda i: (0, 0)),
        ],
        out_shape=[
            jax.ShapeDtypeStruct((N, C), _F32),
            jax.ShapeDtypeStruct((1, C), _F32),
            jax.ShapeDtypeStruct((1, C), _F32),
        ],
    )(x, W1)


def _k2_body(y_ref, a_ref, c_ref, wq_ref, bq_ref, wk_ref, bk_ref,
             wv_ref, bv_ref, xq_ref, xk_ref, xv_ref):
    x1 = jnp.maximum(y_ref[...] * a_ref[...] + c_ref[...], 0.0)
    xq_ref[...] = jnp.dot(x1, wq_ref[...], preferred_element_type=_F32) + bq_ref[...]
    xk_ref[...] = jnp.dot(x1, wk_ref[...], preferred_element_type=_F32) + bk_ref[...]
    xv_ref[...] = jnp.dot(x1, wv_ref[...], preferred_element_type=_F32) + bv_ref[...]


def _k2(y1, a1, c1, Wq, bq, Wk, bk, Wv, bv):
    row = pl.BlockSpec((RB, C), lambda i: (i, 0))
    cst = pl.BlockSpec((1, C), lambda i: (0, 0))
    mat = pl.BlockSpec((C, C), lambda i: (0, 0))
    return pl.pallas_call(
        _k2_body,
        grid=(NRB,),
        in_specs=[row, cst, cst, mat, cst, mat, cst, mat, cst],
        out_specs=[row, row, row],
        out_shape=[jax.ShapeDtypeStruct((N, C), _F32)] * 3,
    )(y1, a1, c1, Wq, bq.reshape(1, C), Wk, bk.reshape(1, C), Wv, bv.reshape(1, C))


def _k3_body(pr_ref, pc_ref, ord_ref, idx_ref, bd_ref, d_scr):
    j = pl.program_id(1)
    prow = pr_ref[...]
    pcol = pc_ref[...]
    dot = lax.dot_general(prow.astype(jnp.bfloat16), pcol.astype(jnp.bfloat16),
                          (((1,), (1,)), ((), ())),
                          preferred_element_type=_F32)
    pn_r = jnp.sum(prow * prow, axis=1, keepdims=True)
    pn_c = lax.dot_general(jnp.ones((1, 16), _F32), pcol * pcol,
                           (((1,), (1,)), ((), ())), preferred_element_type=_F32,
                           precision=lax.Precision.HIGHEST)
    d2 = pn_r + pn_c - 2.0 * dot
    perm = ord_ref[...].reshape(1, CW)
    lane8 = lax.broadcasted_iota(jnp.int32, (RB, K), 1)

    @pl.when(j == 0)
    def _():
        bd = jnp.full((RB, K), jnp.inf, _F32)
        bi = jnp.full((RB, K), _BIG, jnp.int32)
        d = d2
        for _ in range(K):
            m = jnp.min(d, axis=1, keepdims=True)
            vi = jnp.min(jnp.where(d == m, perm, _BIG), axis=1, keepdims=True)
            d = jnp.where((d == m) & (perm == vi), jnp.inf, d)
            worst = jnp.max(bd, axis=1, keepdims=True)
            wi = jnp.max(jnp.where(bd == worst, bi, -1), axis=1, keepdims=True)
            wl = jnp.min(jnp.where((bd == worst) & (bi == wi), lane8, _BIG),
                         axis=1, keepdims=True)
            bd = jnp.where(lane8 == wl, m, bd)
            bi = jnp.where(lane8 == wl, vi, bi)
        bd_ref[...] = bd
        idx_ref[...] = bi

    @pl.when(j > 0)
    def _():
        bd = bd_ref[...]
        bi = idx_ref[...]
        worst = jnp.max(bd, axis=1, keepdims=True)
        wi = jnp.max(jnp.where(bd == worst, bi, -1), axis=1, keepdims=True)
        cand = (d2 < worst) | ((d2 == worst) & (perm < wi))
        cnt = jnp.sum(cand.astype(jnp.int32), axis=1, keepdims=True)
        t = jnp.max(cnt)

        @pl.when(t > 0)
        def _():
            d_scr[...] = d2

            def body(r, carry):
                d = d_scr[...]
                bd2 = bd_ref[...]
                bi2 = idx_ref[...]
                m = jnp.min(d, axis=1, keepdims=True)
                vi2 = jnp.min(jnp.where(d == m, perm, _BIG), axis=1, keepdims=True)
                d_scr[...] = jnp.where((d == m) & (perm == vi2), jnp.inf, d)
                w2 = jnp.max(bd2, axis=1, keepdims=True)
                wj = jnp.max(jnp.where(bd2 == w2, bi2, -1), axis=1, keepdims=True)
                ins = (m < w2) | ((m == w2) & (vi2 < wj))
                wl2 = jnp.min(jnp.where((bd2 == w2) & (bi2 == wj), lane8, _BIG),
                              axis=1, keepdims=True)
                sel = (lane8 == wl2) & ins
                bd_ref[...] = jnp.where(sel, m, bd2)
                idx_ref[...] = jnp.where(sel, vi2, bi2)
                return 0

            lax.fori_loop(0, t, body, 0)


def _col(i, j):
    off = (j + 1) // 2 * (2 * (j % 2) - 1)
    return (i // (CW // RB) + off) % NCT


def _k3(p_s_pad, ordT):
    return pl.pallas_call(
        _k3_body,
        grid=(NRB, NCT),
        in_specs=[
            pl.BlockSpec((RB, 16), lambda i, j: (i, 0)),
            pl.BlockSpec((CW, 16), lambda i, j: (_col(i, j), 0)),
            pl.BlockSpec((1, 1, CW), lambda i, j: (_col(i, j), 0, 0)),
        ],
        out_specs=pl.BlockSpec((RB, K), lambda i, j: (i, 0)),
        out_shape=jax.ShapeDtypeStruct((N, K), jnp.int32),
        scratch_shapes=[pltpu.VMEM((RB, K), _F32), pltpu.VMEM((RB, CW), _F32)],
    )(p_s_pad, p_s_pad, ordT)


def _gather_rows(table, idx_flat, D):
    info = plsc.get_sparse_core_info()
    nw = info.num_cores * info.num_subcores
    b_per_w = B_PAD // nw
    nch = b_per_w // CHUNK

    mesh = plsc.VectorSubcoreMesh(core_axis_name="c", subcore_axis_name="s")

    @functools.partial(
        pl.kernel, mesh=mesh,
        out_type=jax.ShapeDtypeStruct((B_PAD, D), _F32),
        scratch_types=[
            pltpu.VMEM((nch, CHUNK), jnp.int32),
            pltpu.VMEM((2, CHUNK, D), _F32),
            pltpu.SemaphoreType.DMA,
            pltpu.SemaphoreType.DMA,
        ],
    )
    def gk(table_hbm, idx_hbm, out_hbm, idx_v, bufs, sem0, sem1):
        wid = lax.axis_index("s") * info.num_cores + lax.axis_index("c")
        base = wid * b_per_w
        for c in range(nch):
            pltpu.sync_copy(idx_hbm.at[pl.ds(base + c * CHUNK, CHUNK)], idx_v.at[c])
        sems = [sem0, sem1]
        prev = None
        for c in range(nch):
            cur = pltpu.async_copy(table_hbm.at[idx_v.at[c]], bufs.at[c % 2],
                                   sems[c % 2])
            if prev is not None:
                pcp, pc = prev
                pcp.wait()
                pltpu.sync_copy(bufs.at[pc % 2],
                                out_hbm.at[pl.ds(base + pc * CHUNK, CHUNK)])
            prev = (cur, c)
        pcp, pc = prev
        pcp.wait()
        pltpu.sync_copy(bufs.at[pc % 2], out_hbm.at[pl.ds(base + pc * CHUNK, CHUNK)])

    return gk(table, idx_flat)


def _pr_block(pg, pblk, w1p, b1p, ap, cp, w2p, b2p):
    d = (pg[:, :16].reshape(RB, K, 16) - pblk[:, None, :]).reshape(GB, 16)
    h = jnp.dot(d, w1p, preferred_element_type=_F32) + b1p
    h = jnp.maximum(h * ap + cp, 0.0)
    return jnp.dot(h, w2p, preferred_element_type=_F32) + b2p


def _k45_body(pg_ref, p_ref, w1p_ref, b1p_ref, s_ref, q_ref):
    i = pl.program_id(0)
    d = (pg_ref[...][:, :16].reshape(RB, K, 16)
         - p_ref[...][:, None, :]).reshape(GB, 16)
    pr1 = jnp.dot(d, w1p_ref[...], preferred_element_type=_F32) + b1p_ref[...]

    @pl.when(i == 0)
    def _():
        s_ref[...] = jnp.zeros_like(s_ref)
        q_ref[...] = jnp.zeros_like(q_ref)

    s_ref[...] += jnp.sum(pr1, axis=0, keepdims=True)
    q_ref[...] += jnp.sum(pr1 * pr1, axis=0, keepdims=True)


def _k45(pg, p_pad, w1p, b1p):
    return pl.pallas_call(
        _k45_body,
        grid=(NRB,),
        in_specs=[
            pl.BlockSpec((GB, 128), lambda i: (i, 0)),
            pl.BlockSpec((RB, 16), lambda i: (i, 0)),
            pl.BlockSpec((16, 16), lambda i: (0, 0)),
            pl.BlockSpec((1, 16), lambda i: (0, 0)),
        ],
        out_specs=[pl.BlockSpec((1, 16), lambda i: (0, 0))] * 2,
        out_shape=[jax.ShapeDtypeStruct((1, 16), _F32)] * 2,
    )(pg, p_pad, w1p, b1p)


def _k5_body(xkg_ref, pg_ref, p_ref, xq_ref, w1p_ref, b1p_ref, ap_ref, cp_ref,
             w2p_ref, b2p_ref, w0_ref, s_ref, q_ref):
    i = pl.program_id(0)
    pr = _pr_block(pg_ref[...], p_ref[...], w1p_ref[...], b1p_ref[...],
                   ap_ref[...], cp_ref[...], w2p_ref[...], b2p_ref[...])
    w0 = (xkg_ref[...].reshape(RB, K, C) - xq_ref[...][:, None, :]
          + pr.reshape(RB, K, C)).reshape(GB, C)
    w0_ref[...] = w0

    @pl.when(i == 0)
    def _():
        s_ref[...] = jnp.zeros_like(s_ref)
        q_ref[...] = jnp.zeros_like(q_ref)

    s_ref[...] += jnp.sum(w0, axis=0, keepdims=True)
    q_ref[...] += jnp.sum(w0 * w0, axis=0, keepdims=True)


def _k5(xkg, pg, p_pad, xq, w1p, b1p, ap, cp, w2p, b2p):
    return pl.pallas_call(
        _k5_body,
        grid=(NRB,),
        in_specs=[
            pl.BlockSpec((GB, C), lambda i: (i, 0)),
            pl.BlockSpec((GB, 128), lambda i: (i, 0)),
            pl.BlockSpec((RB, 16), lambda i: (i, 0)),
            pl.BlockSpec((RB, C), lambda i: (i, 0)),
            pl.BlockSpec((16, 16), lambda i: (0, 0)),
            pl.BlockSpec((1, 16), lambda i: (0, 0)),
            pl.BlockSpec((1, 16), lambda i: (0, 0)),
            pl.BlockSpec((1, 16), lambda i: (0, 0)),
            pl.BlockSpec((16, C), lambda i: (0, 0)),
            pl.BlockSpec((1, C), lambda i: (0, 0)),
        ],
        out_specs=[
            pl.BlockSpec((GB, C), lambda i: (i, 0)),
            pl.BlockSpec((1, C), lambda i: (0, 0)),
            pl.BlockSpec((1, C), lambda i: (0, 0)),
        ],
        out_shape=[
            jax.ShapeDtypeStruct((B_PAD, C), _F32),
            jax.ShapeDtypeStruct((1, C), _F32),
            jax.ShapeDtypeStruct((1, C), _F32),
        ],
    )(xkg, pg, p_pad, xq, w1p, b1p, ap, cp, w2p, b2p)


def _k6_body(w0_ref, a_ref, c_ref, w_ref, b_ref, w1_ref, s_ref, q_ref):
    i = pl.program_id(0)
    h = jnp.maximum(w0_ref[...] * a_ref[...] + c_ref[...], 0.0)
    w1 = jnp.dot(h, w_ref[...], preferred_element_type=_F32) + b_ref[...]
    w1_ref[...] = w1

    @pl.when(i == 0)
    def _():
        s_ref[...] = jnp.zeros_like(s_ref)
        q_ref[...] = jnp.zeros_like(q_ref)

    s_ref[...] += jnp.sum(w1, axis=0, keepdims=True)
    q_ref[...] += jnp.sum(w1 * w1, axis=0, keepdims=True)


def _k6(w0, aw, cw, Ww1, bw1):
    return pl.pallas_call(
        _k6_body,
        grid=(NRB,),
        in_specs=[
            pl.BlockSpec((GB, C), lambda i: (i, 0)),
            pl.BlockSpec((1, C), lambda i: (0, 0)),
            pl.BlockSpec((1, C), lambda i: (0, 0)),
            pl.BlockSpec((C, CS), lambda i: (0, 0)),
            pl.BlockSpec((1, CS), lambda i: (0, 0)),
        ],
        out_specs=[
            pl.BlockSpec((GB, CS), lambda i: (i, 0)),
            pl.BlockSpec((1, CS), lambda i: (0, 0)),
            pl.BlockSpec((1, CS), lambda i: (0, 0)),
        ],
        out_shape=[
            jax.ShapeDtypeStruct((N * K, CS), _F32),
            jax.ShapeDtypeStruct((1, CS), _F32),
            jax.ShapeDtypeStruct((1, CS), _F32),
        ],
    )(w0, aw, cw, Ww1, bw1.reshape(1, CS))


def _k7_body(w1_ref, xvg_ref, pg_ref, p_ref, a2_ref, c2_ref, ww2_ref, bw2_ref,
             w1p_ref, b1p_ref, ap_ref, cp_ref, w2p_ref, b2p_ref,
             agg_ref, s_ref, q_ref):
    i = pl.program_id(0)
    h = jnp.maximum(w1_ref[...] * a2_ref[...] + c2_ref[...], 0.0)
    w2 = jnp.dot(h, ww2_ref[...], preferred_element_type=_F32) + bw2_ref[...]
    w3 = w2.reshape(RB, K, CS)
    m = jnp.max(w3, axis=1, keepdims=True)
    e = jnp.exp(w3 - m)
    sm = e / jnp.sum(e, axis=1, keepdims=True)
    pr = _pr_block(pg_ref[...], p_ref[...], w1p_ref[...], b1p_ref[...],
                   ap_ref[...], cp_ref[...], w2p_ref[...], b2p_ref[...])
    v0 = xvg_ref[...].reshape(RB, K, C) + pr.reshape(RB, K, C)
    wrep = jnp.concatenate([sm] * S, axis=2)
    agg = jnp.sum(v0 * wrep, axis=1)
    agg_ref[...] = agg

    @pl.when(i == 0)
    def _():
        s_ref[...] = jnp.zeros_like(s_ref)
        q_ref[...] = jnp.zeros_like(q_ref)

    s_ref[...] += jnp.sum(agg, axis=0, keepdims=True)
    q_ref[...] += jnp.sum(agg * agg, axis=0, keepdims=True)


def _k7(w1, xvg, pg, p_pad, a2, c2, Ww2, bw2, w1p, b1p, ap, cp, w2p, b2p):
    return pl.pallas_call(
        _k7_body,
        grid=(NRB,),
        in_specs=[
            pl.BlockSpec((GB, CS), lambda i: (i, 0)),
            pl.BlockSpec((GB, C), lambda i: (i, 0)),
            pl.BlockSpec((GB, 128), lambda i: (i, 0)),
            pl.BlockSpec((RB, 16), lambda i: (i, 0)),
            pl.BlockSpec((1, CS), lambda i: (0, 0)),
            pl.BlockSpec((1, CS), lambda i: (0, 0)),
            pl.BlockSpec((CS, CS), lambda i: (0, 0)),
            pl.BlockSpec((1, CS), lambda i: (0, 0)),
            pl.BlockSpec((16, 16), lambda i: (0, 0)),
            pl.BlockSpec((1, 16), lambda i: (0, 0)),
            pl.BlockSpec((1, 16), lambda i: (0, 0)),
            pl.BlockSpec((1, 16), lambda i: (0, 0)),
            pl.BlockSpec((16, C), lambda i: (0, 0)),
            pl.BlockSpec((1, C), lambda i: (0, 0)),
        ],
        out_specs=[
            pl.BlockSpec((RB, C), lambda i: (i, 0)),
            pl.BlockSpec((1, C), lambda i: (0, 0)),
            pl.BlockSpec((1, C), lambda i: (0, 0)),
        ],
        out_shape=[
            jax.ShapeDtypeStruct((N, C), _F32),
            jax.ShapeDtypeStruct((1, C), _F32),
            jax.ShapeDtypeStruct((1, C), _F32),
        ],
    )(w1, xvg, pg, p_pad, a2, c2, Ww2, bw2.reshape(1, CS),
      w1p, b1p, ap, cp, w2p, b2p)


def _k8_body(agg_ref, a_ref, c_ref, w_ref, y_ref, s_ref, q_ref):
    i = pl.program_id(0)
    x2 = jnp.maximum(agg_ref[...] * a_ref[...] + c_ref[...], 0.0)
    y3 = jnp.dot(x2, w_ref[...], preferred_element_type=_F32)
    y_ref[...] = y3

    @pl.when(i == 0)
    def _():
        s_ref[...] = jnp.zeros_like(s_ref)
        q_ref[...] = jnp.zeros_like(q_ref)

    s_ref[...] += jnp.sum(y3, axis=0, keepdims=True)
    q_ref[...] += jnp.sum(y3 * y3, axis=0, keepdims=True)


def _k8(agg, a2, c2, W3):
    row = pl.BlockSpec((RB, C), lambda i: (i, 0))
    cst = pl.BlockSpec((1, C), lambda i: (0, 0))
    return pl.pallas_call(
        _k8_body,
        grid=(NRB,),
        in_specs=[row, cst, cst, pl.BlockSpec((C, C), lambda i: (0, 0))],
        out_specs=[row, cst, cst],
        out_shape=[
            jax.ShapeDtypeStruct((N, C), _F32),
            jax.ShapeDtypeStruct((1, C), _F32),
            jax.ShapeDtypeStruct((1, C), _F32),
        ],
    )(agg, a2, c2, W3)


def _k9_body(y_ref, x_ref, a_ref, c_ref, o_ref):
    o_ref[...] = jnp.maximum(y_ref[...] * a_ref[...] + c_ref[...] + x_ref[...], 0.0)


def _k9(y3, x, a3, c3):
    row = pl.BlockSpec((RB, C), lambda i: (i, 0))
    cst = pl.BlockSpec((1, C), lambda i: (0, 0))
    return pl.pallas_call(
        _k9_body,
        grid=(NRB,),
        in_specs=[row, row, cst, cst],
        out_specs=row,
        out_shape=jax.ShapeDtypeStruct((N, C), _F32),
    )(y3, x, a3, c3)


def kernel(p, x, o, W1, bn1_g, bn1_b, Wq, bq, Wk, bk, Wv, bv, Wp1, bp1,
           bnp_g, bnp_b, Wp2, bp2, bnw1_g, bnw1_b, Ww1, bw1, bnw2_g, bnw2_b,
           Ww2, bw2, bn2_g, bn2_b, W3, bn3_g, bn3_b):
    p_pad = jnp.pad(p, ((0, 0), (0, 13)))

    y1, s1, q1 = _k1(x, W1)
    a1, c1 = _affine(s1, q1, N, bn1_g, bn1_b)
    xq, xk, xv = _k2(y1, a1, c1, Wq, bq, Wk, bk, Wv, bv)

    lo = jnp.min(p, axis=0)
    hi = jnp.max(p, axis=0)
    q = jnp.clip(((p - lo) / (hi - lo + 1e-9) * 1024.0).astype(jnp.int32),
                 0, 1023)
    code = jnp.zeros((N,), jnp.int32)
    for b in range(10):
        for a in range(3):
            code = code | (((q[:, a] >> b) & 1) << (3 * b + a))
    ord_ = jnp.argsort(code).astype(jnp.int32)
    p_s_pad = jnp.pad(p[ord_], ((0, 0), (0, 13)))
    ordT = ord_.reshape(NCT, 1, CW)
    idx_sorted = _k3(p_s_pad, ordT)
    idx = jnp.zeros((N, K), jnp.int32).at[ord_].set(idx_sorted)
    idx_flat = jnp.pad(idx.reshape(-1), (0, B_PAD - N * K))

    xkg = _gather_rows(xk, idx_flat, C)
    xvg = _gather_rows(xv, idx_flat, C)
    pg = _gather_rows(jnp.pad(p, ((0, 0), (0, 125))), idx_flat, 128)

    w1p = jnp.zeros((16, 16), _F32).at[:3, :3].set(Wp1)
    b1p = jnp.zeros((1, 16), _F32).at[0, :3].set(bp1)
    gp = jnp.ones((16,), _F32).at[:3].set(bnp_g)
    bp = jnp.zeros((16,), _F32).at[:3].set(bnp_b)
    w2p = jnp.zeros((16, C), _F32).at[:3, :].set(Wp2)
    b2p = bp2.reshape(1, C)

    sp, qp = _k45(pg, p_pad, w1p, b1p)
    ap, cp = _affine(sp, qp, N * K, gp.reshape(1, 16), bp.reshape(1, 16))

    w0, sw0, qw0 = _k5(xkg, pg, p_pad, xq, w1p, b1p, ap, cp, w2p, b2p)
    aw0, cw0 = _affine(sw0, qw0, N * K, bnw1_g.reshape(1, C), bnw1_b.reshape(1, C))

    w1a, sw1, qw1 = _k6(w0, aw0, cw0, Ww1, bw1)
    aw1, cw1 = _affine(sw1, qw1, N * K, bnw2_g.reshape(1, CS), bnw2_b.reshape(1, CS))

    agg, sa, qa = _k7(w1a, xvg, pg, p_pad, aw1, cw1, Ww2, bw2,
                      w1p, b1p, ap, cp, w2p, b2p)
    a2, c2 = _affine(sa, qa, N, bn2_g.reshape(1, C), bn2_b.reshape(1, C))

    y3, s3, q3 = _k8(agg, a2, c2, W3)
    a3, c3 = _affine(s3, q3, N, bn3_g.reshape(1, C), bn3_b.reshape(1, C))
    return _k9(y3, x, a3, c3)


# speedup vs baseline: 2.5129x; 1.0682x over previous
import functools

import jax
import jax.numpy as jnp
from jax import lax
from jax.experimental import pallas as pl
from jax.experimental.pallas import tpu as pltpu
from jax.experimental.pallas import tpu_sc as plsc

N = 10000
C = 256
K = 8
S = 8
CS = 32
EPS = 1e-5

RB = 200
NRB = N // RB
CW = 1000
NCT = N // CW
GB = RB * K
B_PAD = 81920
CHUNK = 128

_F32 = jnp.float32
_BIG = 2**30


def _affine(s, q, count, g, b):
    m = s / count
    v = q / count - m * m
    a = g * lax.rsqrt(v + EPS)
    return a, b - m * a


def _k1_body(x_ref, w_ref, y_ref, s_ref, q_ref):
    i = pl.program_id(0)
    y = jnp.dot(x_ref[...], w_ref[...], preferred_element_type=_F32)
    y_ref[...] = y

    @pl.when(i == 0)
    def _():
        s_ref[...] = jnp.zeros_like(s_ref)
        q_ref[...] = jnp.zeros_like(q_ref)

    s_ref[...] += jnp.sum(y, axis=0, keepdims=True)
    q_ref[...] += jnp.sum(y * y, axis=0, keepdims=True)


def _k1(x, W1):
    return pl.pallas_call(
        _k1_body,
        grid=(NRB,),
        in_specs=[
            pl.BlockSpec((RB, C), lambda i: (i, 0)),
            pl.BlockSpec((C, C), lambda i: (0, 0)),
        ],
        out_specs=[
            pl.BlockSpec((RB, C), lambda i: (i, 0)),
            pl.BlockSpec((1, C), lambda i: (0, 0)),
            pl.BlockSpec((1, C), lambda i: (0, 0)),
        ],
        out_shape=[
            jax.ShapeDtypeStruct((N, C), _F32),
            jax.ShapeDtypeStruct((1, C), _F32),
            jax.ShapeDtypeStruct((1, C), _F32),
        ],
    )(x, W1)


def _k2_body(y_ref, a_ref, c_ref, wq_ref, bq_ref, wk_ref, bk_ref,
             wv_ref, bv_ref, xq_ref, xk_ref, xv_ref):
    x1 = jnp.maximum(y_ref[...] * a_ref[...] + c_ref[...], 0.0)
    xq_ref[...] = jnp.dot(x1, wq_ref[...], preferred_element_type=_F32) + bq_ref[...]
    xk_ref[...] = jnp.dot(x1, wk_ref[...], preferred_element_type=_F32) + bk_ref[...]
    xv_ref[...] = jnp.dot(x1, wv_ref[...], preferred_element_type=_F32) + bv_ref[...]


def _k2(y1, a1, c1, Wq, bq, Wk, bk, Wv, bv):
    row = pl.BlockSpec((RB, C), lambda i: (i, 0))
    cst = pl.BlockSpec((1, C), lambda i: (0, 0))
    mat = pl.BlockSpec((C, C), lambda i: (0, 0))
    return pl.pallas_call(
        _k2_body,
        grid=(NRB,),
        in_specs=[row, cst, cst, mat, cst, mat, cst, mat, cst],
        out_specs=[row, row, row],
        out_shape=[jax.ShapeDtypeStruct((N, C), _F32)] * 3,
    )(y1, a1, c1, Wq, bq.reshape(1, C), Wk, bk.reshape(1, C), Wv, bv.reshape(1, C))


def _k3_body(pr_ref, pc_ref, ord_ref, idx_ref, bd_ref, bi_ref, d_scr):
    j = pl.program_id(1)
    prow = pr_ref[...]
    pcol = pc_ref[...]
    dot = lax.dot_general(pcol.astype(jnp.bfloat16), prow.astype(jnp.bfloat16),
                          (((1,), (1,)), ((), ())),
                          preferred_element_type=_F32)
    pn_r = lax.dot_general(jnp.ones((1, 16), _F32), prow * prow,
                           (((1,), (1,)), ((), ())), preferred_element_type=_F32,
                           precision=lax.Precision.HIGHEST)
    pn_c = lax.dot_general(pcol * pcol, jnp.ones((1, 16), _F32),
                           (((1,), (1,)), ((), ())), preferred_element_type=_F32,
                           precision=lax.Precision.HIGHEST)
    d2 = pn_c + pn_r - 2.0 * dot
    perm = ord_ref[...].reshape(CW, 1)
    sub8 = lax.broadcasted_iota(jnp.int32, (K, RB), 0)
    inf = jnp.inf

    def extract(d, bd, bi):
        m = jnp.min(d, axis=0, keepdims=True)
        vi = jnp.min(jnp.where(d == m, perm, inf), axis=0, keepdims=True)
        dn = jnp.where((d == m) & (perm == vi), inf, d)
        worst = jnp.max(bd, axis=0, keepdims=True)
        wi = jnp.max(jnp.where(bd == worst, bi, -1.0), axis=0, keepdims=True)
        ins = (m < worst) | ((m == worst) & (vi < wi))
        wl = jnp.min(jnp.where((bd == worst) & (bi == wi), sub8, _BIG),
                     axis=0, keepdims=True)
        sel = (sub8 == wl) & ins
        return dn, jnp.where(sel, m, bd), jnp.where(sel, vi, bi)

    @pl.when(j == 0)
    def _():
        bd = jnp.full((K, RB), inf, _F32)
        bi = jnp.full((K, RB), 3e9, _F32)
        d = d2
        for _ in range(K):
            d, bd, bi = extract(d, bd, bi)
        bd_ref[...] = bd
        bi_ref[...] = bi
        idx_ref[...] = bi.astype(jnp.int32)[None]

    @pl.when(j > 0)
    def _():
        bd = bd_ref[...]
        bi = bi_ref[...]
        worst = jnp.max(bd, axis=0, keepdims=True)
        wi = jnp.max(jnp.where(bd == worst, bi, -1.0), axis=0, keepdims=True)
        cand = (d2 < worst) | ((d2 == worst) & (perm < wi))
        cnt = jnp.sum(cand.astype(_F32), axis=0, keepdims=True)
        t = jnp.max(cnt).astype(jnp.int32)

        @pl.when(t > 0)
        def _():
            d_scr[...] = d2

            def body(r, carry):
                dn, bd2, bi2 = extract(d_scr[...], bd_ref[...], bi_ref[...])
                d_scr[...] = dn
                bd_ref[...] = bd2
                bi_ref[...] = bi2
                return 0

            lax.fori_loop(0, t, body, 0)
            idx_ref[...] = bi_ref[...].astype(jnp.int32)[None]


def _col(i, j):
    off = (j + 1) // 2 * (2 * (j % 2) - 1)
    return (i // (CW // RB) + off) % NCT


def _k3(p_s_pad, ordF):
    return pl.pallas_call(
        _k3_body,
        grid=(NRB, NCT),
        in_specs=[
            pl.BlockSpec((RB, 16), lambda i, j: (i, 0)),
            pl.BlockSpec((CW, 16), lambda i, j: (_col(i, j), 0)),
            pl.BlockSpec((1, CW, 1), lambda i, j: (_col(i, j), 0, 0)),
        ],
        out_specs=pl.BlockSpec((1, K, RB), lambda i, j: (i, 0, 0)),
        out_shape=jax.ShapeDtypeStruct((NRB, K, RB), jnp.int32),
        scratch_shapes=[pltpu.VMEM((K, RB), _F32), pltpu.VMEM((K, RB), _F32),
                        pltpu.VMEM((CW, RB), _F32)],
    )(p_s_pad, p_s_pad, ordF)


def _gather_rows(table, idx_flat, D):
    info = plsc.get_sparse_core_info()
    nw = info.num_cores * info.num_subcores
    b_per_w = B_PAD // nw
    nch = b_per_w // CHUNK

    mesh = plsc.VectorSubcoreMesh(core_axis_name="c", subcore_axis_name="s")

    @functools.partial(
        pl.kernel, mesh=mesh,
        out_type=jax.ShapeDtypeStruct((B_PAD, D), _F32),
        scratch_types=[
            pltpu.VMEM((nch, CHUNK), jnp.int32),
            pltpu.VMEM((2, CHUNK, D), _F32),
            pltpu.SemaphoreType.DMA,
            pltpu.SemaphoreType.DMA,
        ],
    )
    def gk(table_hbm, idx_hbm, out_hbm, idx_v, bufs, sem0, sem1):
        wid = lax.axis_index("s") * info.num_cores + lax.axis_index("c")
        base = wid * b_per_w
        for c in range(nch):
            pltpu.sync_copy(idx_hbm.at[pl.ds(base + c * CHUNK, CHUNK)], idx_v.at[c])
        sems = [sem0, sem1]
        prev = None
        for c in range(nch):
            cur = pltpu.async_copy(table_hbm.at[idx_v.at[c]], bufs.at[c % 2],
                                   sems[c % 2])
            if prev is not None:
                pcp, pc = prev
                pcp.wait()
                pltpu.sync_copy(bufs.at[pc % 2],
                                out_hbm.at[pl.ds(base + pc * CHUNK, CHUNK)])
            prev = (cur, c)
        pcp, pc = prev
        pcp.wait()
        pltpu.sync_copy(bufs.at[pc % 2], out_hbm.at[pl.ds(base + pc * CHUNK, CHUNK)])

    return gk(table, idx_flat)


def _pr_block(pg, pblk, w1p, b1p, ap, cp, w2p, b2p):
    d = (pg[:, :16].reshape(RB, K, 16) - pblk[:, None, :]).reshape(GB, 16)
    h = jnp.dot(d, w1p, preferred_element_type=_F32) + b1p
    h = jnp.maximum(h * ap + cp, 0.0)
    return jnp.dot(h, w2p, preferred_element_type=_F32) + b2p


def _k45_body(pg_ref, p_ref, w1p_ref, b1p_ref, s_ref, q_ref):
    i = pl.program_id(0)
    d = (pg_ref[...][:, :16].reshape(RB, K, 16)
         - p_ref[...][:, None, :]).reshape(GB, 16)
    pr1 = jnp.dot(d, w1p_ref[...], preferred_element_type=_F32) + b1p_ref[...]

    @pl.when(i == 0)
    def _():
        s_ref[...] = jnp.zeros_like(s_ref)
        q_ref[...] = jnp.zeros_like(q_ref)

    s_ref[...] += jnp.sum(pr1, axis=0, keepdims=True)
    q_ref[...] += jnp.sum(pr1 * pr1, axis=0, keepdims=True)


def _k45(pg, p_pad, w1p, b1p):
    return pl.pallas_call(
        _k45_body,
        grid=(NRB,),
        in_specs=[
            pl.BlockSpec((GB, 128), lambda i: (i, 0)),
            pl.BlockSpec((RB, 16), lambda i: (i, 0)),
            pl.BlockSpec((16, 16), lambda i: (0, 0)),
            pl.BlockSpec((1, 16), lambda i: (0, 0)),
        ],
        out_specs=[pl.BlockSpec((1, 16), lambda i: (0, 0))] * 2,
        out_shape=[jax.ShapeDtypeStruct((1, 16), _F32)] * 2,
    )(pg, p_pad, w1p, b1p)


def _k5_body(xkg_ref, pg_ref, p_ref, xq_ref, w1p_ref, b1p_ref, ap_ref, cp_ref,
             w2p_ref, b2p_ref, w0_ref, s_ref, q_ref):
    i = pl.program_id(0)
    pr = _pr_block(pg_ref[...], p_ref[...], w1p_ref[...], b1p_ref[...],
                   ap_ref[...], cp_ref[...], w2p_ref[...], b2p_ref[...])
    w0 = (xkg_ref[...].reshape(RB, K, C) - xq_ref[...][:, None, :]
          + pr.reshape(RB, K, C)).reshape(GB, C)
    w0_ref[...] = w0

    @pl.when(i == 0)
    def _():
        s_ref[...] = jnp.zeros_like(s_ref)
        q_ref[...] = jnp.zeros_like(q_ref)

    s_ref[...] += jnp.sum(w0, axis=0, keepdims=True)
    q_ref[...] += jnp.sum(w0 * w0, axis=0, keepdims=True)


def _k5(xkg, pg, p_pad, xq, w1p, b1p, ap, cp, w2p, b2p):
    return pl.pallas_call(
        _k5_body,
        grid=(NRB,),
        in_specs=[
            pl.BlockSpec((GB, C), lambda i: (i, 0)),
            pl.BlockSpec((GB, 128), lambda i: (i, 0)),
            pl.BlockSpec((RB, 16), lambda i: (i, 0)),
            pl.BlockSpec((RB, C), lambda i: (i, 0)),
            pl.BlockSpec((16, 16), lambda i: (0, 0)),
            pl.BlockSpec((1, 16), lambda i: (0, 0)),
            pl.BlockSpec((1, 16), lambda i: (0, 0)),
            pl.BlockSpec((1, 16), lambda i: (0, 0)),
            pl.BlockSpec((16, C), lambda i: (0, 0)),
            pl.BlockSpec((1, C), lambda i: (0, 0)),
        ],
        out_specs=[
            pl.BlockSpec((GB, C), lambda i: (i, 0)),
            pl.BlockSpec((1, C), lambda i: (0, 0)),
            pl.BlockSpec((1, C), lambda i: (0, 0)),
        ],
        out_shape=[
            jax.ShapeDtypeStruct((B_PAD, C), _F32),
            jax.ShapeDtypeStruct((1, C), _F32),
            jax.ShapeDtypeStruct((1, C), _F32),
        ],
    )(xkg, pg, p_pad, xq, w1p, b1p, ap, cp, w2p, b2p)


def _k6_body(w0_ref, a_ref, c_ref, w_ref, b_ref, w1_ref, s_ref, q_ref):
    i = pl.program_id(0)
    h = jnp.maximum(w0_ref[...] * a_ref[...] + c_ref[...], 0.0)
    w1 = jnp.dot(h, w_ref[...], preferred_element_type=_F32) + b_ref[...]
    w1_ref[...] = w1

    @pl.when(i == 0)
    def _():
        s_ref[...] = jnp.zeros_like(s_ref)
        q_ref[...] = jnp.zeros_like(q_ref)

    s_ref[...] += jnp.sum(w1, axis=0, keepdims=True)
    q_ref[...] += jnp.sum(w1 * w1, axis=0, keepdims=True)


def _k6(w0, aw, cw, Ww1, bw1):
    return pl.pallas_call(
        _k6_body,
        grid=(NRB,),
        in_specs=[
            pl.BlockSpec((GB, C), lambda i: (i, 0)),
            pl.BlockSpec((1, C), lambda i: (0, 0)),
            pl.BlockSpec((1, C), lambda i: (0, 0)),
            pl.BlockSpec((C, CS), lambda i: (0, 0)),
            pl.BlockSpec((1, CS), lambda i: (0, 0)),
        ],
        out_specs=[
            pl.BlockSpec((GB, CS), lambda i: (i, 0)),
            pl.BlockSpec((1, CS), lambda i: (0, 0)),
            pl.BlockSpec((1, CS), lambda i: (0, 0)),
        ],
        out_shape=[
            jax.ShapeDtypeStruct((N * K, CS), _F32),
            jax.ShapeDtypeStruct((1, CS), _F32),
            jax.ShapeDtypeStruct((1, CS), _F32),
        ],
    )(w0, aw, cw, Ww1, bw1.reshape(1, CS))


def _k7_body(w1_ref, xvg_ref, pg_ref, p_ref, a2_ref, c2_ref, ww2_ref, bw2_ref,
             w1p_ref, b1p_ref, ap_ref, cp_ref, w2p_ref, b2p_ref,
             agg_ref, s_ref, q_ref):
    i = pl.program_id(0)
    h = jnp.maximum(w1_ref[...] * a2_ref[...] + c2_ref[...], 0.0)
    w2 = jnp.dot(h, ww2_ref[...], preferred_element_type=_F32) + bw2_ref[...]
    w3 = w2.reshape(RB, K, CS)
    m = jnp.max(w3, axis=1, keepdims=True)
    e = jnp.exp(w3 - m)
    sm = e / jnp.sum(e, axis=1, keepdims=True)
    pr = _pr_block(pg_ref[...], p_ref[...], w1p_ref[...], b1p_ref[...],
                   ap_ref[...], cp_ref[...], w2p_ref[...], b2p_ref[...])
    v0 = xvg_ref[...].reshape(RB, K, C) + pr.reshape(RB, K, C)
    wrep = jnp.concatenate([sm] * S, axis=2)
    agg = jnp.sum(v0 * wrep, axis=1)
    agg_ref[...] = agg

    @pl.when(i == 0)
    def _():
        s_ref[...] = jnp.zeros_like(s_ref)
        q_ref[...] = jnp.zeros_like(q_ref)

    s_ref[...] += jnp.sum(agg, axis=0, keepdims=True)
    q_ref[...] += jnp.sum(agg * agg, axis=0, keepdims=True)


def _k7(w1, xvg, pg, p_pad, a2, c2, Ww2, bw2, w1p, b1p, ap, cp, w2p, b2p):
    return pl.pallas_call(
        _k7_body,
        grid=(NRB,),
        in_specs=[
            pl.BlockSpec((GB, CS), lambda i: (i, 0)),
            pl.BlockSpec((GB, C), lambda i: (i, 0)),
            pl.BlockSpec((GB, 128), lambda i: (i, 0)),
            pl.BlockSpec((RB, 16), lambda i: (i, 0)),
            pl.BlockSpec((1, CS), lambda i: (0, 0)),
            pl.BlockSpec((1, CS), lambda i: (0, 0)),
            pl.BlockSpec((CS, CS), lambda i: (0, 0)),
            pl.BlockSpec((1, CS), lambda i: (0, 0)),
            pl.BlockSpec((16, 16), lambda i: (0, 0)),
            pl.BlockSpec((1, 16), lambda i: (0, 0)),
            pl.BlockSpec((1, 16), lambda i: (0, 0)),
            pl.BlockSpec((1, 16), lambda i: (0, 0)),
            pl.BlockSpec((16, C), lambda i: (0, 0)),
            pl.BlockSpec((1, C), lambda i: (0, 0)),
        ],
        out_specs=[
            pl.BlockSpec((RB, C), lambda i: (i, 0)),
            pl.BlockSpec((1, C), lambda i: (0, 0)),
            pl.BlockSpec((1, C), lambda i: (0, 0)),
        ],
        out_shape=[
            jax.ShapeDtypeStruct((N, C), _F32),
            jax.ShapeDtypeStruct((1, C), _F32),
            jax.ShapeDtypeStruct((1, C), _F32),
        ],
    )(w1, xvg, pg, p_pad, a2, c2, Ww2, bw2.reshape(1, CS),
      w1p, b1p, ap, cp, w2p, b2p)


def _k8_body(agg_ref, a_ref, c_ref, w_ref, y_ref, s_ref, q_ref):
    i = pl.program_id(0)
    x2 = jnp.maximum(agg_ref[...] * a_ref[...] + c_ref[...], 0.0)
    y3 = jnp.dot(x2, w_ref[...], preferred_element_type=_F32)
    y_ref[...] = y3

    @pl.when(i == 0)
    def _():
        s_ref[...] = jnp.zeros_like(s_ref)
        q_ref[...] = jnp.zeros_like(q_ref)

    s_ref[...] += jnp.sum(y3, axis=0, keepdims=True)
    q_ref[...] += jnp.sum(y3 * y3, axis=0, keepdims=True)


def _k8(agg, a2, c2, W3):
    row = pl.BlockSpec((RB, C), lambda i: (i, 0))
    cst = pl.BlockSpec((1, C), lambda i: (0, 0))
    return pl.pallas_call(
        _k8_body,
        grid=(NRB,),
        in_specs=[row, cst, cst, pl.BlockSpec((C, C), lambda i: (0, 0))],
        out_specs=[row, cst, cst],
        out_shape=[
            jax.ShapeDtypeStruct((N, C), _F32),
            jax.ShapeDtypeStruct((1, C), _F32),
            jax.ShapeDtypeStruct((1, C), _F32),
        ],
    )(agg, a2, c2, W3)


def _k9_body(y_ref, x_ref, a_ref, c_ref, o_ref):
    o_ref[...] = jnp.maximum(y_ref[...] * a_ref[...] + c_ref[...] + x_ref[...], 0.0)


def _k9(y3, x, a3, c3):
    row = pl.BlockSpec((RB, C), lambda i: (i, 0))
    cst = pl.BlockSpec((1, C), lambda i: (0, 0))
    return pl.pallas_call(
        _k9_body,
        grid=(NRB,),
        in_specs=[row, row, cst, cst],
        out_specs=row,
        out_shape=jax.ShapeDtypeStruct((N, C), _F32),
    )(y3, x, a3, c3)


def kernel(p, x, o, W1, bn1_g, bn1_b, Wq, bq, Wk, bk, Wv, bv, Wp1, bp1,
           bnp_g, bnp_b, Wp2, bp2, bnw1_g, bnw1_b, Ww1, bw1, bnw2_g, bnw2_b,
           Ww2, bw2, bn2_g, bn2_b, W3, bn3_g, bn3_b):
    p_pad = jnp.pad(p, ((0, 0), (0, 13)))

    y1, s1, q1 = _k1(x, W1)
    a1, c1 = _affine(s1, q1, N, bn1_g, bn1_b)
    xq, xk, xv = _k2(y1, a1, c1, Wq, bq, Wk, bk, Wv, bv)

    lo = jnp.min(p, axis=0)
    hi = jnp.max(p, axis=0)
    q = jnp.clip(((p - lo) / (hi - lo + 1e-9) * 1024.0).astype(jnp.int32),
                 0, 1023)
    code = jnp.zeros((N,), jnp.int32)
    for b in range(10):
        for a in range(3):
            code = code | (((q[:, a] >> b) & 1) << (3 * b + a))
    ord_ = jnp.argsort(code).astype(jnp.int32)
    p_s_pad = jnp.pad(p[ord_], ((0, 0), (0, 13)))
    ordF = ord_.astype(_F32).reshape(NCT, CW, 1)
    idx_sorted = _k3(p_s_pad, ordF)
    idx_sorted = idx_sorted.transpose(0, 2, 1).reshape(N, K)
    idx = jnp.zeros((N, K), jnp.int32).at[ord_].set(idx_sorted)
    idx_flat = jnp.pad(idx.reshape(-1), (0, B_PAD - N * K))

    xkg = _gather_rows(xk, idx_flat, C)
    xvg = _gather_rows(xv, idx_flat, C)
    pg = _gather_rows(jnp.pad(p, ((0, 0), (0, 125))), idx_flat, 128)

    w1p = jnp.zeros((16, 16), _F32).at[:3, :3].set(Wp1)
    b1p = jnp.zeros((1, 16), _F32).at[0, :3].set(bp1)
    gp = jnp.ones((16,), _F32).at[:3].set(bnp_g)
    bp = jnp.zeros((16,), _F32).at[:3].set(bnp_b)
    w2p = jnp.zeros((16, C), _F32).at[:3, :].set(Wp2)
    b2p = bp2.reshape(1, C)

    sp, qp = _k45(pg, p_pad, w1p, b1p)
    ap, cp = _affine(sp, qp, N * K, gp.reshape(1, 16), bp.reshape(1, 16))

    w0, sw0, qw0 = _k5(xkg, pg, p_pad, xq, w1p, b1p, ap, cp, w2p, b2p)
    aw0, cw0 = _affine(sw0, qw0, N * K, bnw1_g.reshape(1, C), bnw1_b.reshape(1, C))

    w1a, sw1, qw1 = _k6(w0, aw0, cw0, Ww1, bw1)
    aw1, cw1 = _affine(sw1, qw1, N * K, bnw2_g.reshape(1, CS), bnw2_b.reshape(1, CS))

    agg, sa, qa = _k7(w1a, xvg, pg, p_pad, aw1, cw1, Ww2, bw2,
                      w1p, b1p, ap, cp, w2p, b2p)
    a2, c2 = _affine(sa, qa, N, bn2_g.reshape(1, C), bn2_b.reshape(1, C))

    y3, s3, q3 = _k8(agg, a2, c2, W3)
    a3, c3 = _affine(s3, q3, N, bn3_g.reshape(1, C), bn3_b.reshape(1, C))
    return _k9(y3, x, a3, c3)


# speedup vs baseline: 2.8710x; 1.1425x over previous
import functools

import jax
import jax.numpy as jnp
from jax import lax
from jax.experimental import pallas as pl
from jax.experimental.pallas import tpu as pltpu
from jax.experimental.pallas import tpu_sc as plsc

N = 10000
C = 256
K = 8
S = 8
CS = 32
EPS = 1e-5

RB = 200
NRB = N // RB
CW = 400
NCT = N // CW
GB = RB * K
B_PAD = 81920
CHUNK = 128

_F32 = jnp.float32
_BIG = 2**30


def _affine(s, q, count, g, b):
    m = s / count
    v = q / count - m * m
    a = g * lax.rsqrt(v + EPS)
    return a, b - m * a


def _k1_body(x_ref, w_ref, y_ref, s_ref, q_ref):
    i = pl.program_id(0)
    y = jnp.dot(x_ref[...], w_ref[...], preferred_element_type=_F32)
    y_ref[...] = y

    @pl.when(i == 0)
    def _():
        s_ref[...] = jnp.zeros_like(s_ref)
        q_ref[...] = jnp.zeros_like(q_ref)

    s_ref[...] += jnp.sum(y, axis=0, keepdims=True)
    q_ref[...] += jnp.sum(y * y, axis=0, keepdims=True)


def _k1(x, W1):
    return pl.pallas_call(
        _k1_body,
        grid=(NRB,),
        in_specs=[
            pl.BlockSpec((RB, C), lambda i: (i, 0)),
            pl.BlockSpec((C, C), lambda i: (0, 0)),
        ],
        out_specs=[
            pl.BlockSpec((RB, C), lambda i: (i, 0)),
            pl.BlockSpec((1, C), lambda i: (0, 0)),
            pl.BlockSpec((1, C), lambda i: (0, 0)),
        ],
        out_shape=[
            jax.ShapeDtypeStruct((N, C), _F32),
            jax.ShapeDtypeStruct((1, C), _F32),
            jax.ShapeDtypeStruct((1, C), _F32),
        ],
    )(x, W1)


def _k2_body(y_ref, a_ref, c_ref, wq_ref, bq_ref, wk_ref, bk_ref,
             wv_ref, bv_ref, xq_ref, xk_ref, xv_ref):
    x1 = jnp.maximum(y_ref[...] * a_ref[...] + c_ref[...], 0.0)
    xq_ref[...] = jnp.dot(x1, wq_ref[...], preferred_element_type=_F32) + bq_ref[...]
    xk_ref[...] = jnp.dot(x1, wk_ref[...], preferred_element_type=_F32) + bk_ref[...]
    xv_ref[...] = jnp.dot(x1, wv_ref[...], preferred_element_type=_F32) + bv_ref[...]


def _k2(y1, a1, c1, Wq, bq, Wk, bk, Wv, bv):
    row = pl.BlockSpec((RB, C), lambda i: (i, 0))
    cst = pl.BlockSpec((1, C), lambda i: (0, 0))
    mat = pl.BlockSpec((C, C), lambda i: (0, 0))
    return pl.pallas_call(
        _k2_body,
        grid=(NRB,),
        in_specs=[row, cst, cst, mat, cst, mat, cst, mat, cst],
        out_specs=[row, row, row],
        out_shape=[jax.ShapeDtypeStruct((N, C), _F32)] * 3,
    )(y1, a1, c1, Wq, bq.reshape(1, C), Wk, bk.reshape(1, C), Wv, bv.reshape(1, C))


def _k3_body(pr_ref, pc_ref, ord_ref, idx_ref, bd_ref, bi_ref):
    j = pl.program_id(1)
    prow = pr_ref[...]
    pcol = pc_ref[...]
    dot = lax.dot_general(pcol.astype(jnp.bfloat16), prow.astype(jnp.bfloat16),
                          (((1,), (1,)), ((), ())),
                          preferred_element_type=_F32)
    pn_r = lax.dot_general(jnp.ones((1, 16), _F32), prow * prow,
                           (((1,), (1,)), ((), ())), preferred_element_type=_F32,
                           precision=lax.Precision.HIGHEST)
    pn_c = lax.dot_general(pcol * pcol, jnp.ones((1, 16), _F32),
                           (((1,), (1,)), ((), ())), preferred_element_type=_F32,
                           precision=lax.Precision.HIGHEST)
    d2 = pn_c + pn_r - 2.0 * dot
    perm = ord_ref[...].reshape(CW, 1)
    sub8 = lax.broadcasted_iota(jnp.int32, (K, RB), 0)
    inf = jnp.inf

    def extract(d, bd, bi):
        m = jnp.min(d, axis=0, keepdims=True)
        vi = jnp.min(jnp.where(d == m, perm, inf), axis=0, keepdims=True)
        dn = jnp.where((d == m) & (perm == vi), inf, d)
        worst = jnp.max(bd, axis=0, keepdims=True)
        wi = jnp.max(jnp.where(bd == worst, bi, -1.0), axis=0, keepdims=True)
        ins = (m < worst) | ((m == worst) & (vi < wi))
        wl = jnp.min(jnp.where((bd == worst) & (bi == wi), sub8, _BIG),
                     axis=0, keepdims=True)
        sel = (sub8 == wl) & ins
        return dn, jnp.where(sel, m, bd), jnp.where(sel, vi, bi)

    @pl.when(j == 0)
    def _():
        bd = jnp.full((K, RB), inf, _F32)
        bi = jnp.full((K, RB), 3e9, _F32)
        d = d2
        for _ in range(K):
            d, bd, bi = extract(d, bd, bi)
        bd_ref[...] = bd
        bi_ref[...] = bi
        idx_ref[...] = bi.astype(jnp.int32)[None]

    @pl.when(j > 0)
    def _():
        worst0 = jnp.max(bd_ref[...], axis=0, keepdims=True)
        m0 = jnp.min(d2, axis=0, keepdims=True)
        flag = jnp.max(jnp.where(m0 <= worst0, 1.0, 0.0))

        @pl.when(flag > 0.0)
        def _():
            d = d2
            bd = bd_ref[...]
            bi = bi_ref[...]
            for _ in range(K):
                d, bd, bi = extract(d, bd, bi)
            bd_ref[...] = bd
            bi_ref[...] = bi
            idx_ref[...] = bi.astype(jnp.int32)[None]


def _col(i, j):
    off = (j + 1) // 2 * (2 * (j % 2) - 1)
    return ((i * RB) // CW + off) % NCT


def _k3(p_s_pad, ordF):
    return pl.pallas_call(
        _k3_body,
        grid=(NRB, NCT),
        in_specs=[
            pl.BlockSpec((RB, 16), lambda i, j: (i, 0)),
            pl.BlockSpec((CW, 16), lambda i, j: (_col(i, j), 0)),
            pl.BlockSpec((1, CW, 1), lambda i, j: (_col(i, j), 0, 0)),
        ],
        out_specs=pl.BlockSpec((1, K, RB), lambda i, j: (i, 0, 0)),
        out_shape=jax.ShapeDtypeStruct((NRB, K, RB), jnp.int32),
        scratch_shapes=[pltpu.VMEM((K, RB), _F32), pltpu.VMEM((K, RB), _F32)],
    )(p_s_pad, p_s_pad, ordF)


def _gather_rows(table, idx_flat, D):
    info = plsc.get_sparse_core_info()
    nw = info.num_cores * info.num_subcores
    b_per_w = B_PAD // nw
    nch = b_per_w // CHUNK

    mesh = plsc.VectorSubcoreMesh(core_axis_name="c", subcore_axis_name="s")

    @functools.partial(
        pl.kernel, mesh=mesh,
        out_type=jax.ShapeDtypeStruct((B_PAD, D), _F32),
        scratch_types=[
            pltpu.VMEM((nch, CHUNK), jnp.int32),
            pltpu.VMEM((2, CHUNK, D), _F32),
            pltpu.SemaphoreType.DMA,
            pltpu.SemaphoreType.DMA,
        ],
    )
    def gk(table_hbm, idx_hbm, out_hbm, idx_v, bufs, sem0, sem1):
        wid = lax.axis_index("s") * info.num_cores + lax.axis_index("c")
        base = wid * b_per_w
        for c in range(nch):
            pltpu.sync_copy(idx_hbm.at[pl.ds(base + c * CHUNK, CHUNK)], idx_v.at[c])
        sems = [sem0, sem1]
        prev = None
        for c in range(nch):
            cur = pltpu.async_copy(table_hbm.at[idx_v.at[c]], bufs.at[c % 2],
                                   sems[c % 2])
            if prev is not None:
                pcp, pc = prev
                pcp.wait()
                pltpu.sync_copy(bufs.at[pc % 2],
                                out_hbm.at[pl.ds(base + pc * CHUNK, CHUNK)])
            prev = (cur, c)
        pcp, pc = prev
        pcp.wait()
        pltpu.sync_copy(bufs.at[pc % 2], out_hbm.at[pl.ds(base + pc * CHUNK, CHUNK)])

    return gk(table, idx_flat)


def _pr_block(pg, pblk, w1p, b1p, ap, cp, w2p, b2p):
    d = (pg[:, :16].reshape(RB, K, 16) - pblk[:, None, :]).reshape(GB, 16)
    h = jnp.dot(d, w1p, preferred_element_type=_F32) + b1p
    h = jnp.maximum(h * ap + cp, 0.0)
    return jnp.dot(h, w2p, preferred_element_type=_F32) + b2p


def _k45_body(pg_ref, p_ref, w1p_ref, b1p_ref, s_ref, q_ref):
    i = pl.program_id(0)
    d = (pg_ref[...][:, :16].reshape(RB, K, 16)
         - p_ref[...][:, None, :]).reshape(GB, 16)
    pr1 = jnp.dot(d, w1p_ref[...], preferred_element_type=_F32) + b1p_ref[...]

    @pl.when(i == 0)
    def _():
        s_ref[...] = jnp.zeros_like(s_ref)
        q_ref[...] = jnp.zeros_like(q_ref)

    s_ref[...] += jnp.sum(pr1, axis=0, keepdims=True)
    q_ref[...] += jnp.sum(pr1 * pr1, axis=0, keepdims=True)


def _k45(pg, p_pad, w1p, b1p):
    return pl.pallas_call(
        _k45_body,
        grid=(NRB,),
        in_specs=[
            pl.BlockSpec((GB, 128), lambda i: (i, 0)),
            pl.BlockSpec((RB, 16), lambda i: (i, 0)),
            pl.BlockSpec((16, 16), lambda i: (0, 0)),
            pl.BlockSpec((1, 16), lambda i: (0, 0)),
        ],
        out_specs=[pl.BlockSpec((1, 16), lambda i: (0, 0))] * 2,
        out_shape=[jax.ShapeDtypeStruct((1, 16), _F32)] * 2,
    )(pg, p_pad, w1p, b1p)


def _k5_body(xkg_ref, pg_ref, p_ref, xq_ref, w1p_ref, b1p_ref, ap_ref, cp_ref,
             w2p_ref, b2p_ref, w0_ref, s_ref, q_ref):
    i = pl.program_id(0)
    pr = _pr_block(pg_ref[...], p_ref[...], w1p_ref[...], b1p_ref[...],
                   ap_ref[...], cp_ref[...], w2p_ref[...], b2p_ref[...])
    w0 = (xkg_ref[...].reshape(RB, K, C) - xq_ref[...][:, None, :]
          + pr.reshape(RB, K, C)).reshape(GB, C)
    w0_ref[...] = w0

    @pl.when(i == 0)
    def _():
        s_ref[...] = jnp.zeros_like(s_ref)
        q_ref[...] = jnp.zeros_like(q_ref)

    s_ref[...] += jnp.sum(w0, axis=0, keepdims=True)
    q_ref[...] += jnp.sum(w0 * w0, axis=0, keepdims=True)


def _k5(xkg, pg, p_pad, xq, w1p, b1p, ap, cp, w2p, b2p):
    return pl.pallas_call(
        _k5_body,
        grid=(NRB,),
        in_specs=[
            pl.BlockSpec((GB, C), lambda i: (i, 0)),
            pl.BlockSpec((GB, 128), lambda i: (i, 0)),
            pl.BlockSpec((RB, 16), lambda i: (i, 0)),
            pl.BlockSpec((RB, C), lambda i: (i, 0)),
            pl.BlockSpec((16, 16), lambda i: (0, 0)),
            pl.BlockSpec((1, 16), lambda i: (0, 0)),
            pl.BlockSpec((1, 16), lambda i: (0, 0)),
            pl.BlockSpec((1, 16), lambda i: (0, 0)),
            pl.BlockSpec((16, C), lambda i: (0, 0)),
            pl.BlockSpec((1, C), lambda i: (0, 0)),
        ],
        out_specs=[
            pl.BlockSpec((GB, C), lambda i: (i, 0)),
            pl.BlockSpec((1, C), lambda i: (0, 0)),
            pl.BlockSpec((1, C), lambda i: (0, 0)),
        ],
        out_shape=[
            jax.ShapeDtypeStruct((B_PAD, C), _F32),
            jax.ShapeDtypeStruct((1, C), _F32),
            jax.ShapeDtypeStruct((1, C), _F32),
        ],
    )(xkg, pg, p_pad, xq, w1p, b1p, ap, cp, w2p, b2p)


def _k6_body(w0_ref, a_ref, c_ref, w_ref, b_ref, w1_ref, s_ref, q_ref):
    i = pl.program_id(0)
    h = jnp.maximum(w0_ref[...] * a_ref[...] + c_ref[...], 0.0)
    w1 = jnp.dot(h, w_ref[...], preferred_element_type=_F32) + b_ref[...]
    w1_ref[...] = w1

    @pl.when(i == 0)
    def _():
        s_ref[...] = jnp.zeros_like(s_ref)
        q_ref[...] = jnp.zeros_like(q_ref)

    s_ref[...] += jnp.sum(w1, axis=0, keepdims=True)
    q_ref[...] += jnp.sum(w1 * w1, axis=0, keepdims=True)


def _k6(w0, aw, cw, Ww1, bw1):
    return pl.pallas_call(
        _k6_body,
        grid=(NRB,),
        in_specs=[
            pl.BlockSpec((GB, C), lambda i: (i, 0)),
            pl.BlockSpec((1, C), lambda i: (0, 0)),
            pl.BlockSpec((1, C), lambda i: (0, 0)),
            pl.BlockSpec((C, CS), lambda i: (0, 0)),
            pl.BlockSpec((1, CS), lambda i: (0, 0)),
        ],
        out_specs=[
            pl.BlockSpec((GB, CS), lambda i: (i, 0)),
            pl.BlockSpec((1, CS), lambda i: (0, 0)),
            pl.BlockSpec((1, CS), lambda i: (0, 0)),
        ],
        out_shape=[
            jax.ShapeDtypeStruct((N * K, CS), _F32),
            jax.ShapeDtypeStruct((1, CS), _F32),
            jax.ShapeDtypeStruct((1, CS), _F32),
        ],
    )(w0, aw, cw, Ww1, bw1.reshape(1, CS))


def _k7_body(w1_ref, xvg_ref, pg_ref, p_ref, a2_ref, c2_ref, ww2_ref, bw2_ref,
             w1p_ref, b1p_ref, ap_ref, cp_ref, w2p_ref, b2p_ref,
             agg_ref, s_ref, q_ref):
    i = pl.program_id(0)
    h = jnp.maximum(w1_ref[...] * a2_ref[...] + c2_ref[...], 0.0)
    w2 = jnp.dot(h, ww2_ref[...], preferred_element_type=_F32) + bw2_ref[...]
    w3 = w2.reshape(RB, K, CS)
    m = jnp.max(w3, axis=1, keepdims=True)
    e = jnp.exp(w3 - m)
    sm = e / jnp.sum(e, axis=1, keepdims=True)
    pr = _pr_block(pg_ref[...], p_ref[...], w1p_ref[...], b1p_ref[...],
                   ap_ref[...], cp_ref[...], w2p_ref[...], b2p_ref[...])
    v0 = xvg_ref[...].reshape(RB, K, C) + pr.reshape(RB, K, C)
    wrep = jnp.concatenate([sm] * S, axis=2)
    agg = jnp.sum(v0 * wrep, axis=1)
    agg_ref[...] = agg

    @pl.when(i == 0)
    def _():
        s_ref[...] = jnp.zeros_like(s_ref)
        q_ref[...] = jnp.zeros_like(q_ref)

    s_ref[...] += jnp.sum(agg, axis=0, keepdims=True)
    q_ref[...] += jnp.sum(agg * agg, axis=0, keepdims=True)


def _k7(w1, xvg, pg, p_pad, a2, c2, Ww2, bw2, w1p, b1p, ap, cp, w2p, b2p):
    return pl.pallas_call(
        _k7_body,
        grid=(NRB,),
        in_specs=[
            pl.BlockSpec((GB, CS), lambda i: (i, 0)),
            pl.BlockSpec((GB, C), lambda i: (i, 0)),
            pl.BlockSpec((GB, 128), lambda i: (i, 0)),
            pl.BlockSpec((RB, 16), lambda i: (i, 0)),
            pl.BlockSpec((1, CS), lambda i: (0, 0)),
            pl.BlockSpec((1, CS), lambda i: (0, 0)),
            pl.BlockSpec((CS, CS), lambda i: (0, 0)),
            pl.BlockSpec((1, CS), lambda i: (0, 0)),
            pl.BlockSpec((16, 16), lambda i: (0, 0)),
            pl.BlockSpec((1, 16), lambda i: (0, 0)),
            pl.BlockSpec((1, 16), lambda i: (0, 0)),
            pl.BlockSpec((1, 16), lambda i: (0, 0)),
            pl.BlockSpec((16, C), lambda i: (0, 0)),
            pl.BlockSpec((1, C), lambda i: (0, 0)),
        ],
        out_specs=[
            pl.BlockSpec((RB, C), lambda i: (i, 0)),
            pl.BlockSpec((1, C), lambda i: (0, 0)),
            pl.BlockSpec((1, C), lambda i: (0, 0)),
        ],
        out_shape=[
            jax.ShapeDtypeStruct((N, C), _F32),
            jax.ShapeDtypeStruct((1, C), _F32),
            jax.ShapeDtypeStruct((1, C), _F32),
        ],
    )(w1, xvg, pg, p_pad, a2, c2, Ww2, bw2.reshape(1, CS),
      w1p, b1p, ap, cp, w2p, b2p)


def _k8_body(agg_ref, a_ref, c_ref, w_ref, y_ref, s_ref, q_ref):
    i = pl.program_id(0)
    x2 = jnp.maximum(agg_ref[...] * a_ref[...] + c_ref[...], 0.0)
    y3 = jnp.dot(x2, w_ref[...], preferred_element_type=_F32)
    y_ref[...] = y3

    @pl.when(i == 0)
    def _():
        s_ref[...] = jnp.zeros_like(s_ref)
        q_ref[...] = jnp.zeros_like(q_ref)

    s_ref[...] += jnp.sum(y3, axis=0, keepdims=True)
    q_ref[...] += jnp.sum(y3 * y3, axis=0, keepdims=True)


def _k8(agg, a2, c2, W3):
    row = pl.BlockSpec((RB, C), lambda i: (i, 0))
    cst = pl.BlockSpec((1, C), lambda i: (0, 0))
    return pl.pallas_call(
        _k8_body,
        grid=(NRB,),
        in_specs=[row, cst, cst, pl.BlockSpec((C, C), lambda i: (0, 0))],
        out_specs=[row, cst, cst],
        out_shape=[
            jax.ShapeDtypeStruct((N, C), _F32),
            jax.ShapeDtypeStruct((1, C), _F32),
            jax.ShapeDtypeStruct((1, C), _F32),
        ],
    )(agg, a2, c2, W3)


def _k9_body(y_ref, x_ref, a_ref, c_ref, o_ref):
    o_ref[...] = jnp.maximum(y_ref[...] * a_ref[...] + c_ref[...] + x_ref[...], 0.0)


def _k9(y3, x, a3, c3):
    row = pl.BlockSpec((RB, C), lambda i: (i, 0))
    cst = pl.BlockSpec((1, C), lambda i: (0, 0))
    return pl.pallas_call(
        _k9_body,
        grid=(NRB,),
        in_specs=[row, row, cst, cst],
        out_specs=row,
        out_shape=jax.ShapeDtypeStruct((N, C), _F32),
    )(y3, x, a3, c3)


def kernel(p, x, o, W1, bn1_g, bn1_b, Wq, bq, Wk, bk, Wv, bv, Wp1, bp1,
           bnp_g, bnp_b, Wp2, bp2, bnw1_g, bnw1_b, Ww1, bw1, bnw2_g, bnw2_b,
           Ww2, bw2, bn2_g, bn2_b, W3, bn3_g, bn3_b):
    p_pad = jnp.pad(p, ((0, 0), (0, 13)))

    y1, s1, q1 = _k1(x, W1)
    a1, c1 = _affine(s1, q1, N, bn1_g, bn1_b)
    xq, xk, xv = _k2(y1, a1, c1, Wq, bq, Wk, bk, Wv, bv)

    lo = jnp.min(p, axis=0)
    hi = jnp.max(p, axis=0)
    q = jnp.clip(((p - lo) / (hi - lo + 1e-9) * 1024.0).astype(jnp.int32),
                 0, 1023)
    code = jnp.zeros((N,), jnp.int32)
    for b in range(10):
        for a in range(3):
            code = code | (((q[:, a] >> b) & 1) << (3 * b + a))
    ord_ = jnp.argsort(code).astype(jnp.int32)
    p_s_pad = jnp.pad(p[ord_], ((0, 0), (0, 13)))
    ordF = ord_.astype(_F32).reshape(NCT, CW, 1)
    idx_sorted = _k3(p_s_pad, ordF)
    idx_sorted = idx_sorted.transpose(0, 2, 1).reshape(N, K)
    idx = jnp.zeros((N, K), jnp.int32).at[ord_].set(idx_sorted)
    idx_flat = jnp.pad(idx.reshape(-1), (0, B_PAD - N * K))

    xkg = _gather_rows(xk, idx_flat, C)
    xvg = _gather_rows(xv, idx_flat, C)
    pg = _gather_rows(jnp.pad(p, ((0, 0), (0, 125))), idx_flat, 128)

    w1p = jnp.zeros((16, 16), _F32).at[:3, :3].set(Wp1)
    b1p = jnp.zeros((1, 16), _F32).at[0, :3].set(bp1)
    gp = jnp.ones((16,), _F32).at[:3].set(bnp_g)
    bp = jnp.zeros((16,), _F32).at[:3].set(bnp_b)
    w2p = jnp.zeros((16, C), _F32).at[:3, :].set(Wp2)
    b2p = bp2.reshape(1, C)

    sp, qp = _k45(pg, p_pad, w1p, b1p)
    ap, cp = _affine(sp, qp, N * K, gp.reshape(1, 16), bp.reshape(1, 16))

    w0, sw0, qw0 = _k5(xkg, pg, p_pad, xq, w1p, b1p, ap, cp, w2p, b2p)
    aw0, cw0 = _affine(sw0, qw0, N * K, bnw1_g.reshape(1, C), bnw1_b.reshape(1, C))

    w1a, sw1, qw1 = _k6(w0, aw0, cw0, Ww1, bw1)
    aw1, cw1 = _affine(sw1, qw1, N * K, bnw2_g.reshape(1, CS), bnw2_b.reshape(1, CS))

    agg, sa, qa = _k7(w1a, xvg, pg, p_pad, aw1, cw1, Ww2, bw2,
                      w1p, b1p, ap, cp, w2p, b2p)
    a2, c2 = _affine(sa, qa, N, bn2_g.reshape(1, C), bn2_b.reshape(1, C))

    y3, s3, q3 = _k8(agg, a2, c2, W3)
    a3, c3 = _affine(s3, q3, N, bn3_g.reshape(1, C), bn3_b.reshape(1, C))
    return _k9(y3, x, a3, c3)
